# Initial kernel scaffold; baseline (speedup 1.0000x reference)
import jax
import jax.numpy as jnp
from jax.experimental import pallas as pl


def kernel(edge_index, edge_values, user_emb, item_emb, brand_emb):
    raise NotImplementedError("write your pallas kernel here")


# speedup vs baseline: 6.2674x; 6.2674x over previous
import functools

import jax
import jax.numpy as jnp
from jax import lax
from jax.experimental import pallas as pl
from jax.experimental.pallas import tpu as pltpu
from jax.experimental.pallas import tpu_sc as plsc

NUM_USERS = 6000
NUM_ITEMS = 3500
NUM_BRANDS = 500
N_NODES = NUM_USERS + NUM_ITEMS + NUM_BRANDS
D = 128
N_EDGES = 320000
N_LAYERS = 3

NC = 2
NS = 16
NW = NC * NS
CW = 128
CH = 80
EPT = CH * CW
E_PAD = NW * EPT
N_PAD = 10240
ROWS_PER_TILE = N_PAD // NS
WB = 128

_mesh = plsc.VectorSubcoreMesh(core_axis_name="c", subcore_axis_name="s")


@functools.partial(
    pl.kernel,
    mesh=_mesh,
    out_type=jax.ShapeDtypeStruct((NC, N_PAD, D), jnp.float32),
    scratch_types=[
        pltpu.VMEM((CH, CW), jnp.int32),
        pltpu.VMEM((CH, CW), jnp.int32),
        pltpu.VMEM((CH, CW), jnp.float32),
        pltpu.VMEM((CW, D), jnp.float32),
        pltpu.VMEM_SHARED((N_PAD, D), jnp.float32),
        pltpu.SemaphoreType.DMA,
    ],
)
def _layer_sc(ego_hbm, src_hbm, dst_hbm, val_hbm, out_hbm,
              src_v, dst_v, val_v, rows_v, acc_sh, sem):
    cid = lax.axis_index("c")
    sid = lax.axis_index("s")
    wid = cid * NS + sid

    pltpu.sync_copy(src_hbm.at[wid], src_v)
    pltpu.sync_copy(dst_hbm.at[wid], dst_v)
    pltpu.sync_copy(val_hbm.at[wid], val_v)

    zero = jnp.zeros((16,), jnp.float32)

    @pl.loop(0, CW)
    def _(r):
        for k in range(D // 16):
            rows_v[r, pl.ds(k * 16, 16)] = zero

    base = sid * ROWS_PER_TILE
    for i in range(ROWS_PER_TILE // WB):
        pltpu.sync_copy(rows_v.at[pl.ds(0, WB)],
                        acc_sh.at[pl.ds(base + i * WB, WB)])
    plsc.subcore_barrier()

    @pl.loop(0, CH)
    def _(j):
        pltpu.async_copy(ego_hbm.at[src_v.at[j]], rows_v, sem).wait()

        @pl.loop(0, CW // 16)
        def _(g):
            vvec = val_v[j, pl.ds(g * 16, 16)]
            for l in range(16):
                vv = jnp.full((16,), vvec[l], jnp.float32)
                e = g * 16 + l
                for k in range(D // 16):
                    sl = pl.ds(k * 16, 16)
                    rows_v[e, sl] = rows_v[e, sl] * vv

        pltpu.sync_copy(rows_v, acc_sh.at[dst_v.at[j]], add=True)

    plsc.subcore_barrier()

    for i in range(ROWS_PER_TILE // WB):
        pltpu.sync_copy(acc_sh.at[pl.ds(base + i * WB, WB)],
                        out_hbm.at[cid, pl.ds(base + i * WB, WB)])


def _add2_body(p_ref, o_ref):
    o_ref[...] = p_ref[0] + p_ref[1]


_add2 = pl.pallas_call(
    _add2_body,
    out_shape=jax.ShapeDtypeStruct((N_PAD, D), jnp.float32),
)


def _final_body(e0_ref, e1_ref, e2_ref, p3_ref, o_ref):
    o_ref[...] = 0.25 * (e0_ref[...] + e1_ref[...] + e2_ref[...]
                         + p3_ref[0] + p3_ref[1])


_final = pl.pallas_call(
    _final_body,
    out_shape=jax.ShapeDtypeStruct((N_PAD, D), jnp.float32),
)


@jax.jit
def kernel(edge_index, edge_values, user_emb, item_emb, brand_emb):
    ego0 = jnp.concatenate(
        [user_emb, item_emb, brand_emb,
         jnp.zeros((N_PAD - N_NODES, D), jnp.float32)], axis=0)
    dst = edge_index[0]
    src = edge_index[1]

    pad = E_PAD - N_EDGES
    fill = (jnp.arange(pad, dtype=jnp.int32) * 97) % N_NODES
    src_p = jnp.concatenate([src, fill]).reshape(NW, CH, CW)
    dst_p = jnp.concatenate([dst, fill]).reshape(NW, CH, CW)
    val_p = jnp.concatenate(
        [edge_values, jnp.zeros((pad,), jnp.float32)]).reshape(NW, CH, CW)

    p1 = _layer_sc(ego0, src_p, dst_p, val_p)
    ego1 = _add2(p1)
    p2 = _layer_sc(ego1, src_p, dst_p, val_p)
    ego2 = _add2(p2)
    p3 = _layer_sc(ego2, src_p, dst_p, val_p)
    final = _final(ego0, ego1, ego2, p3)

    final_user = final[:NUM_USERS]
    final_item = final[NUM_USERS:NUM_USERS + NUM_ITEMS]
    return (final_user, final_item, user_emb, item_emb)


# speedup vs baseline: 9.9901x; 1.5940x over previous
import functools

import jax
import jax.numpy as jnp
from jax import lax
from jax.experimental import pallas as pl
from jax.experimental.pallas import tpu as pltpu
from jax.experimental.pallas import tpu_sc as plsc

NUM_USERS = 6000
NUM_ITEMS = 3500
NUM_BRANDS = 500
N_NODES = NUM_USERS + NUM_ITEMS + NUM_BRANDS
D = 128
N_EDGES = 320000
N_LAYERS = 3

NC = 2
NS = 16
NW = NC * NS
CW = 128
CH = 80
EPT = CH * CW
E_PAD = NW * EPT
N_PAD = 10240
ROWS_PER_TILE = N_PAD // NS
WB = 128
SB = 8
NB = CH // SB

_mesh = plsc.VectorSubcoreMesh(core_axis_name="c", subcore_axis_name="s")


@functools.partial(
    pl.kernel,
    mesh=_mesh,
    out_type=jax.ShapeDtypeStruct((NC, N_PAD, D), jnp.float32),
    scratch_types=[
        pltpu.VMEM((2 * SB, CW), jnp.int32),
        pltpu.VMEM((2 * SB, CW), jnp.int32),
        pltpu.VMEM((2 * SB, CW), jnp.float32),
        pltpu.VMEM((CW, D), jnp.float32),
        pltpu.VMEM((CW, D), jnp.float32),
        pltpu.VMEM_SHARED((N_PAD, D), jnp.float32),
        pltpu.SemaphoreType.DMA,
        pltpu.SemaphoreType.DMA,
        pltpu.SemaphoreType.DMA,
    ],
)
def _layer_sc(ego_hbm, src_hbm, dst_hbm, val_hbm, out_hbm,
              src_v, dst_v, val_v, rows_a, rows_b, acc_sh,
              gs0, gs1, sl_sem):
    cid = lax.axis_index("c")
    sid = lax.axis_index("s")
    wid = cid * NS + sid

    def slab_copies(bb, p):
        return [
            pltpu.make_async_copy(h.at[wid, pl.ds(bb * SB, SB)],
                                  v.at[pl.ds(p * SB, SB)], sl_sem)
            for h, v in ((src_hbm, src_v), (dst_hbm, dst_v),
                         (val_hbm, val_v))
        ]

    def stage_start(bb, p):
        for c in slab_copies(bb, p):
            c.start()

    def stage_wait(bb, p):
        for c in slab_copies(bb, p):
            c.wait()

    def gather(srow, buf, sem):
        return pltpu.make_async_copy(ego_hbm.at[src_v.at[srow]], buf, sem)

    def scale(srow, buf):
        @pl.loop(0, CW // 16)
        def _(g):
            vvec = val_v[srow, pl.ds(g * 16, 16)]
            for l in range(16):
                vv = jnp.full((16,), vvec[l], jnp.float32)
                e = g * 16 + l
                for k in range(D // 16):
                    sl = pl.ds(k * 16, 16)
                    buf[e, sl] = buf[e, sl] * vv

    stage_start(0, 0)
    stage_wait(0, 0)

    zero = jnp.zeros((16,), jnp.float32)

    @pl.loop(0, CW)
    def _(r):
        for k in range(D // 16):
            rows_b[r, pl.ds(k * 16, 16)] = zero

    gather(0, rows_a, gs0).start()

    base = sid * ROWS_PER_TILE
    for i in range(ROWS_PER_TILE // WB):
        pltpu.sync_copy(rows_b.at[pl.ds(0, WB)],
                        acc_sh.at[pl.ds(base + i * WB, WB)])
    plsc.subcore_barrier()

    @pl.loop(0, NB, step=2)
    def _(b):
        for off, p in ((0, 0), (1, 1)):
            bb = b + off

            @pl.when(bb + 1 < NB)
            def _():
                stage_start(bb + 1, 1 - p)

            @pl.loop(0, SB, step=2)
            def _(r):
                srow = p * SB + r
                gather(srow, rows_a, gs0).wait()
                gather(srow + 1, rows_b, gs1).start()
                scale(srow, rows_a)
                pltpu.sync_copy(rows_a, acc_sh.at[dst_v.at[srow]], add=True)
                gather(srow + 1, rows_b, gs1).wait()

                @pl.when(r + 2 < SB)
                def _():
                    gather(srow + 2, rows_a, gs0).start()

                @pl.when(r + 2 >= SB)
                def _():
                    @pl.when(bb + 1 < NB)
                    def _():
                        stage_wait(bb + 1, 1 - p)
                        gather((1 - p) * SB, rows_a, gs0).start()

                scale(srow + 1, rows_b)
                pltpu.sync_copy(rows_b, acc_sh.at[dst_v.at[srow + 1]],
                                add=True)

    plsc.subcore_barrier()

    for i in range(ROWS_PER_TILE // WB):
        pltpu.sync_copy(acc_sh.at[pl.ds(base + i * WB, WB)],
                        out_hbm.at[cid, pl.ds(base + i * WB, WB)])


def _add2_body(p_ref, o_ref):
    o_ref[...] = p_ref[0] + p_ref[1]


_add2 = pl.pallas_call(
    _add2_body,
    out_shape=jax.ShapeDtypeStruct((N_PAD, D), jnp.float32),
)


def _final_body(e0_ref, e1_ref, e2_ref, p3_ref, o_ref):
    o_ref[...] = 0.25 * (e0_ref[...] + e1_ref[...] + e2_ref[...]
                         + p3_ref[0] + p3_ref[1])


_final = pl.pallas_call(
    _final_body,
    out_shape=jax.ShapeDtypeStruct((N_PAD, D), jnp.float32),
)


@jax.jit
def kernel(edge_index, edge_values, user_emb, item_emb, brand_emb):
    ego0 = jnp.concatenate(
        [user_emb, item_emb, brand_emb,
         jnp.zeros((N_PAD - N_NODES, D), jnp.float32)], axis=0)
    dst = edge_index[0]
    src = edge_index[1]

    pad = E_PAD - N_EDGES
    fill = (jnp.arange(pad, dtype=jnp.int32) * 97) % N_NODES
    src_p = jnp.concatenate([src, fill]).reshape(NW, CH, CW)
    dst_p = jnp.concatenate([dst, fill]).reshape(NW, CH, CW)
    val_p = jnp.concatenate(
        [edge_values, jnp.zeros((pad,), jnp.float32)]).reshape(NW, CH, CW)

    p1 = _layer_sc(ego0, src_p, dst_p, val_p)
    ego1 = _add2(p1)
    p2 = _layer_sc(ego1, src_p, dst_p, val_p)
    ego2 = _add2(p2)
    p3 = _layer_sc(ego2, src_p, dst_p, val_p)
    final = _final(ego0, ego1, ego2, p3)

    final_user = final[:NUM_USERS]
    final_item = final[NUM_USERS:NUM_USERS + NUM_ITEMS]
    return (final_user, final_item, user_emb, item_emb)


# speedup vs baseline: 10.0232x; 1.0033x over previous
import functools

import jax
import jax.numpy as jnp
from jax import lax
from jax.experimental import pallas as pl
from jax.experimental.pallas import tpu as pltpu
from jax.experimental.pallas import tpu_sc as plsc

NUM_USERS = 6000
NUM_ITEMS = 3500
NUM_BRANDS = 500
N_NODES = NUM_USERS + NUM_ITEMS + NUM_BRANDS
D = 128
N_EDGES = 320000
N_LAYERS = 3

NC = 2
NS = 16
NW = NC * NS
CW = 128
CH = 80
EPT = CH * CW
E_PAD = NW * EPT
N_PAD = 10240
ROWS_PER_TILE = N_PAD // NS
WB = 128
SB = 8
NB = CH // SB

_mesh = plsc.VectorSubcoreMesh(core_axis_name="c", subcore_axis_name="s")


@functools.partial(
    pl.kernel,
    mesh=_mesh,
    out_type=jax.ShapeDtypeStruct((NC, N_PAD, D), jnp.float32),
    scratch_types=[
        pltpu.VMEM((2 * SB, CW), jnp.int32),
        pltpu.VMEM((2 * SB, CW), jnp.int32),
        pltpu.VMEM((2 * SB, CW), jnp.float32),
        pltpu.VMEM((CW, D), jnp.float32),
        pltpu.VMEM((CW, D), jnp.float32),
        pltpu.VMEM_SHARED((N_PAD, D), jnp.float32),
        pltpu.SemaphoreType.DMA,
        pltpu.SemaphoreType.DMA,
        pltpu.SemaphoreType.DMA,
        pltpu.SemaphoreType.DMA,
        pltpu.SemaphoreType.DMA,
    ],
)
def _layer_sc(ego_hbm, src_hbm, dst_hbm, val_hbm, out_hbm,
              src_v, dst_v, val_v, rows_a, rows_b, acc_sh,
              gs0, gs1, sl_sem, ss0, ss1):
    cid = lax.axis_index("c")
    sid = lax.axis_index("s")
    wid = cid * NS + sid

    def slab_copies(bb, p):
        return [
            pltpu.make_async_copy(h.at[wid, pl.ds(bb * SB, SB)],
                                  v.at[pl.ds(p * SB, SB)], sl_sem)
            for h, v in ((src_hbm, src_v), (dst_hbm, dst_v),
                         (val_hbm, val_v))
        ]

    def stage_start(bb, p):
        for c in slab_copies(bb, p):
            c.start()

    def stage_wait(bb, p):
        for c in slab_copies(bb, p):
            c.wait()

    def gather(srow, buf, sem):
        return pltpu.make_async_copy(ego_hbm.at[src_v.at[srow]], buf, sem)

    def scale(srow, buf):
        @pl.loop(0, CW // 16)
        def _(g):
            vvec = val_v[srow, pl.ds(g * 16, 16)]
            for l in range(16):
                vv = jnp.full((16,), vvec[l], jnp.float32)
                e = g * 16 + l
                for k in range(D // 16):
                    sl = pl.ds(k * 16, 16)
                    buf[e, sl] = buf[e, sl] * vv

    stage_start(0, 0)
    stage_wait(0, 0)

    zero = jnp.zeros((16,), jnp.float32)

    @pl.loop(0, CW)
    def _(r):
        for k in range(D // 16):
            rows_b[r, pl.ds(k * 16, 16)] = zero

    gather(0, rows_a, gs0).start()

    base = sid * ROWS_PER_TILE
    for i in range(ROWS_PER_TILE // WB):
        pltpu.sync_copy(rows_b.at[pl.ds(0, WB)],
                        acc_sh.at[pl.ds(base + i * WB, WB)])
    plsc.subcore_barrier()

    def scat_start(srow, buf, sem):
        pltpu.async_copy(buf, acc_sh.at[dst_v.at[srow]], sem, add=True)

    def scat_wait(buf, sem):
        pltpu.make_async_copy(buf, acc_sh.at[dst_v.at[0]], sem).wait()

    @pl.loop(0, NB)
    def _(bb):
        p = bb % 2

        @pl.when(bb + 1 < NB)
        def _():
            stage_start(bb + 1, 1 - p)

        @pl.loop(0, SB, step=2)
        def _(r):
            srow = p * SB + r
            gather(srow, rows_a, gs0).wait()

            @pl.when(bb + r > 0)
            def _():
                scat_wait(rows_b, ss1)

            gather(srow + 1, rows_b, gs1).start()
            scale(srow, rows_a)
            scat_start(srow, rows_a, ss0)
            gather(srow + 1, rows_b, gs1).wait()
            scat_wait(rows_a, ss0)

            @pl.when(r + 2 < SB)
            def _():
                gather(srow + 2, rows_a, gs0).start()

            @pl.when(r + 2 >= SB)
            def _():
                @pl.when(bb + 1 < NB)
                def _():
                    stage_wait(bb + 1, 1 - p)
                    gather((1 - p) * SB, rows_a, gs0).start()

            scale(srow + 1, rows_b)
            scat_start(srow + 1, rows_b, ss1)

    scat_wait(rows_b, ss1)
    plsc.subcore_barrier()

    for i in range(ROWS_PER_TILE // WB):
        pltpu.sync_copy(acc_sh.at[pl.ds(base + i * WB, WB)],
                        out_hbm.at[cid, pl.ds(base + i * WB, WB)])


def _add2_body(p_ref, o_ref):
    o_ref[...] = p_ref[0] + p_ref[1]


_add2 = pl.pallas_call(
    _add2_body,
    out_shape=jax.ShapeDtypeStruct((N_PAD, D), jnp.float32),
)


def _final_body(e0_ref, e1_ref, e2_ref, p3_ref, o_ref):
    o_ref[...] = 0.25 * (e0_ref[...] + e1_ref[...] + e2_ref[...]
                         + p3_ref[0] + p3_ref[1])


_final = pl.pallas_call(
    _final_body,
    out_shape=jax.ShapeDtypeStruct((N_PAD, D), jnp.float32),
)


@jax.jit
def kernel(edge_index, edge_values, user_emb, item_emb, brand_emb):
    ego0 = jnp.concatenate(
        [user_emb, item_emb, brand_emb,
         jnp.zeros((N_PAD - N_NODES, D), jnp.float32)], axis=0)
    dst = edge_index[0]
    src = edge_index[1]

    pad = E_PAD - N_EDGES
    fill = (jnp.arange(pad, dtype=jnp.int32) * 97) % N_NODES
    src_p = jnp.concatenate([src, fill]).reshape(NW, CH, CW)
    dst_p = jnp.concatenate([dst, fill]).reshape(NW, CH, CW)
    val_p = jnp.concatenate(
        [edge_values, jnp.zeros((pad,), jnp.float32)]).reshape(NW, CH, CW)

    p1 = _layer_sc(ego0, src_p, dst_p, val_p)
    ego1 = _add2(p1)
    p2 = _layer_sc(ego1, src_p, dst_p, val_p)
    ego2 = _add2(p2)
    p3 = _layer_sc(ego2, src_p, dst_p, val_p)
    final = _final(ego0, ego1, ego2, p3)

    final_user = final[:NUM_USERS]
    final_item = final[NUM_USERS:NUM_USERS + NUM_ITEMS]
    return (final_user, final_item, user_emb, item_emb)
